# baseline (speedup 1.0000x reference)
import functools

import numpy as np
import jax
import jax.numpy as jnp
from jax import lax
from jax.experimental import pallas as pl
from jax.experimental.pallas import tpu as pltpu
from jax.experimental.pallas import tpu_sc as plsc

B = 4096
DIN = 128
DOUT = 112
NSPL = 16
NT = 50
SB = 256
CANVAS_W = 1024
NCORES = 2
NSUB = 16
NWORK = NCORES * NSUB
SPT = B // NWORK
CHUNK = 16
NCHUNKS = SPT // CHUNK

_LOG2PI = float(np.log(2.0 * np.pi))
ENTROPY_C = float(DOUT * (0.5 + 0.5 * _LOG2PI))
LOGPROB_C = float(DOUT * (-0.5 * _LOG2PI))


def _tc_points_kernel(w_ref, b_ref, x_ref, b0_ref, b1_ref, b2_ref,
                      sample_ref, idx_ref, wgt_ref):
    xb = x_ref[...].astype(jnp.bfloat16)
    mu = lax.dot_general(w_ref[...].astype(jnp.bfloat16), xb,
                         (((1,), (1,)), ((), ())),
                         preferred_element_type=jnp.float32)
    mu = mu + b_ref[...]
    sg = 1.0 / (1.0 + jnp.exp(-mu))
    sample_ref[...] = sg[0:DOUT]
    par = sg[DOUT:2 * DOUT] * 28.0
    p0x = par[0:16]
    p0y = par[16:32]
    p1x = par[32:48]
    p1y = par[48:64]
    p2x = par[64:80]
    p2y = par[80:96]
    wgt_ref[...] = par[96:112] * (-0.003)
    b0 = b0_ref[...]
    b1 = b1_ref[...]
    b2 = b2_ref[...]
    px = (b0 * p0x[:, None, :] + b1 * p1x[:, None, :]) + b2 * p2x[:, None, :]
    py = (b0 * p0y[:, None, :] + b1 * p1y[:, None, :]) + b2 * p2y[:, None, :]
    cx = jnp.round(px)
    cy = jnp.round(py)
    lane = lax.broadcasted_iota(jnp.int32, (1, 1, SB), 2)
    off = (jnp.bitwise_and(lane, 3) * 32
           + jnp.bitwise_and(lax.shift_right_logical(lane, 2), 3) * 4096)
    idx_ref[...] = (cx * 128.0 + cy).astype(jnp.int32) + off


def _tc_fold_kernel(e_ref, sk_ref, lp_ref, en_ref):
    e = e_ref[...]
    zm = jnp.roll(e, 1, axis=2)
    zp = jnp.roll(e, -1, axis=2)
    ylane = jnp.bitwise_and(
        lax.broadcasted_iota(jnp.int32, e.shape, 2), 31)
    ty = (zm + e) + zp
    ty = ty + jnp.where(ylane == 0, e, 0.0)
    ty = ty + jnp.where(ylane == 27, e + 2.0 * zp, 0.0)
    xm = jnp.roll(ty, 1, axis=1)
    xp = jnp.roll(ty, -1, axis=1)
    xi = lax.broadcasted_iota(jnp.int32, e.shape, 1)
    tx = (xm + ty) + xp
    tx = tx + jnp.where(xi == 0, ty, 0.0)
    tx = tx + jnp.where(xi == 27, ty + 2.0 * xp, 0.0)
    sk_ref[...] = jnp.clip(tx[:, 0:28, :] + 0.3, 0.0, 1.0)
    lp_ref[...] = jnp.full((SB,), LOGPROB_C, jnp.float32)
    en_ref[...] = jnp.full((SB,), ENTROPY_C, jnp.float32)


def _make_sc_scatter():
    mesh = plsc.VectorSubcoreMesh(core_axis_name="c", subcore_axis_name="s")

    @functools.partial(
        pl.kernel, mesh=mesh,
        compiler_params=pltpu.CompilerParams(
            needs_layout_passes=False, use_tc_tiling_on_sc=False),
        out_type=jax.ShapeDtypeStruct((B * CANVAS_W,), jnp.float32),
        scratch_types=[
            pltpu.VMEM((2, NSPL, NT, CHUNK), jnp.int32),
            pltpu.VMEM((NSPL, SPT), jnp.float32),
            pltpu.VMEM((2, CHUNK * CANVAS_W), jnp.float32),
            pltpu.SemaphoreType.DMA((2,)),
            pltpu.SemaphoreType.DMA((2,)),
        ],
    )
    def sc_scatter(idx_hbm, w_hbm, out_hbm, idx_v, w_v, canvas_v,
                   idx_sem, out_sem):
        wid = lax.axis_index("s") * NCORES + lax.axis_index("c")
        s_base = wid * SPT
        zeros16 = jnp.zeros((16,), jnp.float32)

        pltpu.sync_copy(w_hbm.at[:, pl.ds(s_base, SPT)], w_v)

        def start_idx(k, buf):
            return pltpu.make_async_copy(
                idx_hbm.at[:, :, pl.ds(s_base + k * CHUNK, CHUNK)],
                idx_v.at[buf], idx_sem.at[buf])

        start_idx(0, 0).start()
        out_handles = [None, None]

        for k in range(NCHUNKS):
            buf = k % 2
            start_idx(k, buf).wait()
            if k + 1 < NCHUNKS:
                start_idx(k + 1, 1 - buf).start()

            if out_handles[buf] is not None:
                for h in out_handles[buf]:
                    h.wait()
                out_handles[buf] = None

            cbuf = canvas_v.at[buf]

            def zero_body(i, c):
                cbuf[pl.ds(i * 16, 16)] = zeros16
                return c
            lax.fori_loop(0, CHUNK * CANVAS_W // 16, zero_body, 0,
                          unroll=8)

            wvs = [w_v[sp, pl.ds(k * CHUNK, CHUNK)] for sp in range(NSPL)]

            def t_body(t, c):
                for sp in range(NSPL):
                    iv = idx_v[buf, sp, t, :]
                    plsc.addupdate_scatter(cbuf, [iv], wvs[sp])
                return c
            lax.fori_loop(0, NT, t_body, 0)

            h = pltpu.make_async_copy(
                cbuf,
                out_hbm.at[pl.ds((s_base + k * CHUNK) * CANVAS_W,
                                 CHUNK * CANVAS_W)],
                out_sem.at[buf])
            h.start()
            out_handles[buf] = [h]

        for hb in out_handles:
            if hb is not None:
                for h in hb:
                    h.wait()

    return sc_scatter


_sc_scatter = _make_sc_scatter()


def kernel(x, W_lin, b_lin):
    wt = W_lin.T
    wperm = wt.reshape(NSPL, 7, DIN).transpose(1, 0, 2).reshape(DOUT, DIN)
    wcat = jnp.concatenate([wt, wperm], axis=0)
    bperm = b_lin.reshape(NSPL, 7).T.reshape(DOUT)
    bcat = jnp.concatenate([b_lin, bperm], axis=0)[:, None]

    t = jnp.linspace(0.0, 1.0, NT)
    b0 = ((1 - t) ** 2).reshape(1, NT, 1)
    b1 = (2 * (1 - t) * t).reshape(1, NT, 1)
    b2 = (t ** 2).reshape(1, NT, 1)

    grid = B // SB
    sample_t, idx_t, wgt_t = pl.pallas_call(
        _tc_points_kernel,
        grid=(grid,),
        in_specs=[
            pl.BlockSpec((2 * DOUT, DIN), lambda i: (0, 0)),
            pl.BlockSpec((2 * DOUT, 1), lambda i: (0, 0)),
            pl.BlockSpec((SB, DIN), lambda i: (i, 0)),
            pl.BlockSpec((1, NT, 1), lambda i: (0, 0, 0)),
            pl.BlockSpec((1, NT, 1), lambda i: (0, 0, 0)),
            pl.BlockSpec((1, NT, 1), lambda i: (0, 0, 0)),
        ],
        out_specs=[
            pl.BlockSpec((DOUT, SB), lambda i: (0, i)),
            pl.BlockSpec((NSPL, NT, SB), lambda i: (0, 0, i)),
            pl.BlockSpec((NSPL, SB), lambda i: (0, i)),
        ],
        out_shape=[
            jax.ShapeDtypeStruct((DOUT, B), jnp.float32),
            jax.ShapeDtypeStruct((NSPL, NT, B), jnp.int32),
            jax.ShapeDtypeStruct((NSPL, B), jnp.float32),
        ],
    )(wcat, bcat, x, b0, b1, b2)

    e4 = _sc_scatter(idx_t, wgt_t).reshape(B // 4, 32, 128)

    skp, log_prob, entropy = pl.pallas_call(
        _tc_fold_kernel,
        grid=(grid,),
        in_specs=[pl.BlockSpec((SB // 4, 32, 128), lambda i: (i, 0, 0))],
        out_specs=[
            pl.BlockSpec((SB // 4, 28, 128), lambda i: (i, 0, 0)),
            pl.BlockSpec((SB,), lambda i: (i,)),
            pl.BlockSpec((SB,), lambda i: (i,)),
        ],
        out_shape=[
            jax.ShapeDtypeStruct((B // 4, 28, 128), jnp.float32),
            jax.ShapeDtypeStruct((B,), jnp.float32),
            jax.ShapeDtypeStruct((B,), jnp.float32),
        ],
    )(e4)

    sketch = (skp.reshape(B // 4, 28, 4, 32).transpose(0, 2, 1, 3)
              .reshape(B, 28, 32)[:, :, 0:28])
    return (sketch, log_prob, entropy, sample_t.T)


# speedup vs baseline: 1.7547x; 1.7547x over previous
import functools

import numpy as np
import jax
import jax.numpy as jnp
from jax import lax
from jax.experimental import pallas as pl
from jax.experimental.pallas import tpu as pltpu
from jax.experimental.pallas import tpu_sc as plsc

B = 4096
DIN = 128
DOUT = 112
NSPL = 16
NT = 50
SB = 256
CANVAS_W = 1024
NCORES = 2
NSUB = 16
NWORK = NCORES * NSUB
SPT = B // NWORK
CHUNK = 16
NCHUNKS = SPT // CHUNK

_LOG2PI = float(np.log(2.0 * np.pi))
ENTROPY_C = float(DOUT * (0.5 + 0.5 * _LOG2PI))
LOGPROB_C = float(DOUT * (-0.5 * _LOG2PI))


def _tc_points_kernel(w_ref, b_ref, x_ref, b0_ref, b1_ref, b2_ref,
                      sample_ref, idx_ref, wgt_ref):
    xb = x_ref[...].astype(jnp.bfloat16)
    mu = lax.dot_general(w_ref[...].astype(jnp.bfloat16), xb,
                         (((1,), (1,)), ((), ())),
                         preferred_element_type=jnp.float32)
    mu = mu + b_ref[...]
    sg = 1.0 / (1.0 + jnp.exp(-mu))
    sample_ref[...] = jnp.transpose(sg[0:DOUT], (1, 0))
    par = sg[DOUT:2 * DOUT] * 28.0
    p0x = par[0:16]
    p0y = par[16:32]
    p1x = par[32:48]
    p1y = par[48:64]
    p2x = par[64:80]
    p2y = par[80:96]
    wgt_ref[...] = par[96:112] * (-0.003)
    b0 = b0_ref[...]
    b1 = b1_ref[...]
    b2 = b2_ref[...]
    px = (b0 * p0x[:, None, :] + b1 * p1x[:, None, :]) + b2 * p2x[:, None, :]
    py = (b0 * p0y[:, None, :] + b1 * p1y[:, None, :]) + b2 * p2y[:, None, :]
    cx = jnp.round(px)
    cy = jnp.round(py)
    lane = lax.broadcasted_iota(jnp.int32, (1, 1, SB), 2)
    off = (jnp.bitwise_and(lane, 3) * 32
           + jnp.bitwise_and(lax.shift_right_logical(lane, 2), 3) * 4096)
    idx_ref[...] = (cx * 128.0 + cy).astype(jnp.int32) + off


def _tc_fold_kernel(e_ref, sk_ref, lp_ref, en_ref):
    e = e_ref[...]
    zm = jnp.roll(e, 1, axis=2)
    zp = jnp.roll(e, -1, axis=2)
    ylane = jnp.bitwise_and(
        lax.broadcasted_iota(jnp.int32, e.shape, 2), 31)
    ty = (zm + e) + zp
    ty = ty + jnp.where(ylane == 0, e, 0.0)
    ty = ty + jnp.where(ylane == 27, e + 2.0 * zp, 0.0)
    xm = jnp.roll(ty, 1, axis=1)
    xp = jnp.roll(ty, -1, axis=1)
    xi = lax.broadcasted_iota(jnp.int32, e.shape, 1)
    tx = (xm + ty) + xp
    tx = tx + jnp.where(xi == 0, ty, 0.0)
    tx = tx + jnp.where(xi == 27, ty + 2.0 * xp, 0.0)
    sk = jnp.clip(tx[:, 0:28, :] + 0.3, 0.0, 1.0)
    parts = [sk[:, :, 32 * i:32 * i + 28] for i in range(4)]
    sk_ref[...] = jnp.stack(parts, axis=1).reshape(SB, 28, 28)
    lp_ref[...] = jnp.full((SB,), LOGPROB_C, jnp.float32)
    en_ref[...] = jnp.full((SB,), ENTROPY_C, jnp.float32)


def _make_sc_scatter():
    mesh = plsc.VectorSubcoreMesh(core_axis_name="c", subcore_axis_name="s")

    @functools.partial(
        pl.kernel, mesh=mesh,
        compiler_params=pltpu.CompilerParams(
            needs_layout_passes=False, use_tc_tiling_on_sc=False),
        out_type=jax.ShapeDtypeStruct((B * CANVAS_W,), jnp.float32),
        scratch_types=[
            pltpu.VMEM((2, NSPL, NT, CHUNK), jnp.int32),
            pltpu.VMEM((NSPL, SPT), jnp.float32),
            pltpu.VMEM((2, CHUNK * CANVAS_W), jnp.float32),
            pltpu.SemaphoreType.DMA((2,)),
            pltpu.SemaphoreType.DMA((2,)),
        ],
    )
    def sc_scatter(idx_hbm, w_hbm, out_hbm, idx_v, w_v, canvas_v,
                   idx_sem, out_sem):
        wid = lax.axis_index("s") * NCORES + lax.axis_index("c")
        s_base = wid * SPT
        zeros16 = jnp.zeros((16,), jnp.float32)

        pltpu.sync_copy(w_hbm.at[:, pl.ds(s_base, SPT)], w_v)

        def start_idx(k, buf):
            return pltpu.make_async_copy(
                idx_hbm.at[:, :, pl.ds(s_base + k * CHUNK, CHUNK)],
                idx_v.at[buf], idx_sem.at[buf])

        start_idx(0, 0).start()
        out_handles = [None, None]

        for k in range(NCHUNKS):
            buf = k % 2
            start_idx(k, buf).wait()
            if k + 1 < NCHUNKS:
                start_idx(k + 1, 1 - buf).start()

            if out_handles[buf] is not None:
                for h in out_handles[buf]:
                    h.wait()
                out_handles[buf] = None

            cbuf = canvas_v.at[buf]

            def zero_body(i, c):
                cbuf[pl.ds(i * 16, 16)] = zeros16
                return c
            lax.fori_loop(0, CHUNK * CANVAS_W // 16, zero_body, 0,
                          unroll=8)

            wvs = [w_v[sp, pl.ds(k * CHUNK, CHUNK)] for sp in range(NSPL)]

            def t_body(t, c):
                for sp in range(NSPL):
                    iv = idx_v[buf, sp, t, :]
                    plsc.addupdate_scatter(cbuf, [iv], wvs[sp])
                return c
            lax.fori_loop(0, NT, t_body, 0)

            h = pltpu.make_async_copy(
                cbuf,
                out_hbm.at[pl.ds((s_base + k * CHUNK) * CANVAS_W,
                                 CHUNK * CANVAS_W)],
                out_sem.at[buf])
            h.start()
            out_handles[buf] = [h]

        for hb in out_handles:
            if hb is not None:
                for h in hb:
                    h.wait()

    return sc_scatter


_sc_scatter = _make_sc_scatter()


def kernel(x, W_lin, b_lin):
    wt = W_lin.T
    wperm = wt.reshape(NSPL, 7, DIN).transpose(1, 0, 2).reshape(DOUT, DIN)
    wcat = jnp.concatenate([wt, wperm], axis=0)
    bperm = b_lin.reshape(NSPL, 7).T.reshape(DOUT)
    bcat = jnp.concatenate([b_lin, bperm], axis=0)[:, None]

    t = jnp.linspace(0.0, 1.0, NT)
    b0 = ((1 - t) ** 2).reshape(1, NT, 1)
    b1 = (2 * (1 - t) * t).reshape(1, NT, 1)
    b2 = (t ** 2).reshape(1, NT, 1)

    grid = B // SB
    sample_t, idx_t, wgt_t = pl.pallas_call(
        _tc_points_kernel,
        grid=(grid,),
        in_specs=[
            pl.BlockSpec((2 * DOUT, DIN), lambda i: (0, 0)),
            pl.BlockSpec((2 * DOUT, 1), lambda i: (0, 0)),
            pl.BlockSpec((SB, DIN), lambda i: (i, 0)),
            pl.BlockSpec((1, NT, 1), lambda i: (0, 0, 0)),
            pl.BlockSpec((1, NT, 1), lambda i: (0, 0, 0)),
            pl.BlockSpec((1, NT, 1), lambda i: (0, 0, 0)),
        ],
        out_specs=[
            pl.BlockSpec((SB, DOUT), lambda i: (i, 0)),
            pl.BlockSpec((NSPL, NT, SB), lambda i: (0, 0, i)),
            pl.BlockSpec((NSPL, SB), lambda i: (0, i)),
        ],
        out_shape=[
            jax.ShapeDtypeStruct((B, DOUT), jnp.float32),
            jax.ShapeDtypeStruct((NSPL, NT, B), jnp.int32),
            jax.ShapeDtypeStruct((NSPL, B), jnp.float32),
        ],
    )(wcat, bcat, x, b0, b1, b2)

    e4 = _sc_scatter(idx_t, wgt_t).reshape(B // 4, 32, 128)

    sketch, log_prob, entropy = pl.pallas_call(
        _tc_fold_kernel,
        grid=(grid,),
        in_specs=[pl.BlockSpec((SB // 4, 32, 128), lambda i: (i, 0, 0))],
        out_specs=[
            pl.BlockSpec((SB, 28, 28), lambda i: (i, 0, 0)),
            pl.BlockSpec((SB,), lambda i: (i,)),
            pl.BlockSpec((SB,), lambda i: (i,)),
        ],
        out_shape=[
            jax.ShapeDtypeStruct((B, 28, 28), jnp.float32),
            jax.ShapeDtypeStruct((B,), jnp.float32),
            jax.ShapeDtypeStruct((B,), jnp.float32),
        ],
    )(e4)

    return (sketch, log_prob, entropy, sample_t)


# speedup vs baseline: 2.0152x; 1.1485x over previous
import functools

import numpy as np
import jax
import jax.numpy as jnp
from jax import lax
from jax.experimental import pallas as pl
from jax.experimental.pallas import tpu as pltpu
from jax.experimental.pallas import tpu_sc as plsc

B = 4096
DIN = 128
DOUT = 112
NSPL = 16
NT = 50
SB = 256
CANVAS_W = 1024
NCORES = 2
NSUB = 16
NWORK = NCORES * NSUB
SPT = B // NWORK
CHUNK = 16
NCHUNKS = SPT // CHUNK

_LOG2PI = float(np.log(2.0 * np.pi))
ENTROPY_C = float(DOUT * (0.5 + 0.5 * _LOG2PI))
LOGPROB_C = float(DOUT * (-0.5 * _LOG2PI))


def _tc_points_kernel(w_ref, b_ref, x_ref, b0_ref, b1_ref, b2_ref,
                      sample_ref, idx_ref, wgt_ref):
    xb = x_ref[...].astype(jnp.bfloat16)
    mu = lax.dot_general(w_ref[...].astype(jnp.bfloat16), xb,
                         (((1,), (1,)), ((), ())),
                         preferred_element_type=jnp.float32)
    mu = mu + b_ref[...]
    sg = 1.0 / (1.0 + jnp.exp(-mu))
    sample_ref[...] = jnp.transpose(sg[0:DOUT], (1, 0))
    par = sg[DOUT:2 * DOUT] * 28.0
    p0x = par[0:16]
    p0y = par[16:32]
    p1x = par[32:48]
    p1y = par[48:64]
    p2x = par[64:80]
    p2y = par[80:96]
    wgt_ref[...] = par[96:112] * (-0.003)
    b0 = b0_ref[...]
    b1 = b1_ref[...]
    b2 = b2_ref[...]
    px = (b0 * p0x[:, None, :] + b1 * p1x[:, None, :]) + b2 * p2x[:, None, :]
    py = (b0 * p0y[:, None, :] + b1 * p1y[:, None, :]) + b2 * p2y[:, None, :]
    cx = jnp.round(px)
    cy = jnp.round(py)
    lane = lax.broadcasted_iota(jnp.int32, (1, 1, SB), 2)
    off = (jnp.bitwise_and(lane, 3) * 32
           + jnp.bitwise_and(lax.shift_right_logical(lane, 2), 3) * 4096)
    idx_ref[...] = (cx * 128.0 + cy).astype(jnp.int32) + off


def _tc_fold_kernel(e_ref, sk_ref, lp_ref, en_ref):
    e = e_ref[...]
    zm = jnp.roll(e, 1, axis=2)
    zp = jnp.roll(e, -1, axis=2)
    ylane = jnp.bitwise_and(
        lax.broadcasted_iota(jnp.int32, e.shape, 2), 31)
    ty = (zm + e) + zp
    ty = ty + jnp.where(ylane == 0, e, 0.0)
    ty = ty + jnp.where(ylane == 27, e + 2.0 * zp, 0.0)
    xm = jnp.roll(ty, 1, axis=1)
    xp = jnp.roll(ty, -1, axis=1)
    xi = lax.broadcasted_iota(jnp.int32, e.shape, 1)
    tx = (xm + ty) + xp
    tx = tx + jnp.where(xi == 0, ty, 0.0)
    tx = tx + jnp.where(xi == 27, ty + 2.0 * xp, 0.0)
    sk = jnp.clip(tx[:, 0:28, :] + 0.3, 0.0, 1.0)
    parts = [sk[:, :, 32 * i:32 * i + 28] for i in range(4)]
    sk_ref[...] = jnp.stack(parts, axis=1).reshape(SB, 28, 28)
    lp_ref[...] = jnp.full((SB,), LOGPROB_C, jnp.float32)
    en_ref[...] = jnp.full((SB,), ENTROPY_C, jnp.float32)


def _make_sc_scatter():
    mesh = plsc.VectorSubcoreMesh(core_axis_name="c", subcore_axis_name="s")

    @functools.partial(
        pl.kernel, mesh=mesh,
        compiler_params=pltpu.CompilerParams(
            needs_layout_passes=False, use_tc_tiling_on_sc=False),
        out_type=jax.ShapeDtypeStruct((B * CANVAS_W,), jnp.float32),
        scratch_types=[
            pltpu.VMEM((2, NSPL, NT, CHUNK), jnp.int32),
            pltpu.VMEM((NSPL, SPT), jnp.float32),
            pltpu.VMEM((2, CHUNK * CANVAS_W), jnp.float32),
            pltpu.SemaphoreType.DMA((2,)),
            pltpu.SemaphoreType.DMA((2,)),
        ],
    )
    def sc_scatter(idx_hbm, w_hbm, out_hbm, idx_v, w_v, canvas_v,
                   idx_sem, out_sem):
        wid = lax.axis_index("s") * NCORES + lax.axis_index("c")
        s_base = wid * SPT
        zeros16 = jnp.zeros((16,), jnp.float32)

        pltpu.sync_copy(w_hbm.at[:, pl.ds(s_base, SPT)], w_v)

        def start_idx(k, buf):
            return pltpu.make_async_copy(
                idx_hbm.at[:, :, pl.ds(s_base + k * CHUNK, CHUNK)],
                idx_v.at[buf], idx_sem.at[buf])

        start_idx(0, 0).start()
        out_handles = [None, None]

        for k in range(NCHUNKS):
            buf = k % 2
            start_idx(k, buf).wait()
            if k + 1 < NCHUNKS:
                start_idx(k + 1, 1 - buf).start()

            if out_handles[buf] is not None:
                for h in out_handles[buf]:
                    h.wait()
                out_handles[buf] = None

            cbuf = canvas_v.at[buf]

            def zero_body(i, c):
                cbuf[pl.ds(i * 16, 16)] = zeros16
                return c
            lax.fori_loop(0, CHUNK * CANVAS_W // 16, zero_body, 0,
                          unroll=8)

            wvs = [w_v[sp, pl.ds(k * CHUNK, CHUNK)] for sp in range(NSPL)]

            def t_body(t, c):
                ivs = [idx_v[buf, sp, t, :] for sp in range(NSPL)]
                for sp in range(NSPL):
                    plsc.addupdate_scatter(cbuf, [ivs[sp]], wvs[sp])
                return c
            lax.fori_loop(0, NT, t_body, 0)

            h = pltpu.make_async_copy(
                cbuf,
                out_hbm.at[pl.ds((s_base + k * CHUNK) * CANVAS_W,
                                 CHUNK * CANVAS_W)],
                out_sem.at[buf])
            h.start()
            out_handles[buf] = [h]

        for hb in out_handles:
            if hb is not None:
                for h in hb:
                    h.wait()

    return sc_scatter


_sc_scatter = _make_sc_scatter()


def kernel(x, W_lin, b_lin):
    wt = W_lin.T
    wperm = wt.reshape(NSPL, 7, DIN).transpose(1, 0, 2).reshape(DOUT, DIN)
    wcat = jnp.concatenate([wt, wperm], axis=0)
    bperm = b_lin.reshape(NSPL, 7).T.reshape(DOUT)
    bcat = jnp.concatenate([b_lin, bperm], axis=0)[:, None]

    t = jnp.linspace(0.0, 1.0, NT)
    b0 = ((1 - t) ** 2).reshape(1, NT, 1)
    b1 = (2 * (1 - t) * t).reshape(1, NT, 1)
    b2 = (t ** 2).reshape(1, NT, 1)

    grid = B // SB
    sample_t, idx_t, wgt_t = pl.pallas_call(
        _tc_points_kernel,
        grid=(grid,),
        in_specs=[
            pl.BlockSpec((2 * DOUT, DIN), lambda i: (0, 0)),
            pl.BlockSpec((2 * DOUT, 1), lambda i: (0, 0)),
            pl.BlockSpec((SB, DIN), lambda i: (i, 0)),
            pl.BlockSpec((1, NT, 1), lambda i: (0, 0, 0)),
            pl.BlockSpec((1, NT, 1), lambda i: (0, 0, 0)),
            pl.BlockSpec((1, NT, 1), lambda i: (0, 0, 0)),
        ],
        out_specs=[
            pl.BlockSpec((SB, DOUT), lambda i: (i, 0)),
            pl.BlockSpec((NSPL, NT, SB), lambda i: (0, 0, i)),
            pl.BlockSpec((NSPL, SB), lambda i: (0, i)),
        ],
        out_shape=[
            jax.ShapeDtypeStruct((B, DOUT), jnp.float32),
            jax.ShapeDtypeStruct((NSPL, NT, B), jnp.int32),
            jax.ShapeDtypeStruct((NSPL, B), jnp.float32),
        ],
    )(wcat, bcat, x, b0, b1, b2)

    e4 = _sc_scatter(idx_t, wgt_t).reshape(B // 4, 32, 128)

    sketch, log_prob, entropy = pl.pallas_call(
        _tc_fold_kernel,
        grid=(grid,),
        in_specs=[pl.BlockSpec((SB // 4, 32, 128), lambda i: (i, 0, 0))],
        out_specs=[
            pl.BlockSpec((SB, 28, 28), lambda i: (i, 0, 0)),
            pl.BlockSpec((SB,), lambda i: (i,)),
            pl.BlockSpec((SB,), lambda i: (i,)),
        ],
        out_shape=[
            jax.ShapeDtypeStruct((B, 28, 28), jnp.float32),
            jax.ShapeDtypeStruct((B,), jnp.float32),
            jax.ShapeDtypeStruct((B,), jnp.float32),
        ],
    )(e4)

    return (sketch, log_prob, entropy, sample_t)


# speedup vs baseline: 3.1637x; 1.5699x over previous
import functools

import numpy as np
import jax
import jax.numpy as jnp
from jax import lax
from jax.experimental import pallas as pl
from jax.experimental.pallas import tpu as pltpu
from jax.experimental.pallas import tpu_sc as plsc

B = 4096
DIN = 128
DOUT = 112
NSPL = 16
NT = 50
SB = 256
CANVAS_W = 1024
NCORES = 2
NSUB = 16
NWORK = NCORES * NSUB
SPT = B // NWORK
CH64 = 64

_LOG2PI = float(np.log(2.0 * np.pi))
ENTROPY_C = float(DOUT * (0.5 + 0.5 * _LOG2PI))
LOGPROB_C = float(DOUT * (-0.5 * _LOG2PI))


def _tc_points_kernel(w_ref, b_ref, x_ref, b0_ref, b1_ref, b2_ref,
                      sample_ref, idx_ref, wgt_ref):
    xb = x_ref[...].astype(jnp.bfloat16)
    mu = lax.dot_general(w_ref[...].astype(jnp.bfloat16), xb,
                         (((1,), (1,)), ((), ())),
                         preferred_element_type=jnp.float32)
    mu = mu + b_ref[...]
    sg = 1.0 / (1.0 + jnp.exp(-mu))
    sample_ref[...] = jnp.transpose(sg[0:DOUT], (1, 0))
    par = sg[DOUT:2 * DOUT] * 28.0
    p0x = par[0:16]
    p0y = par[16:32]
    p1x = par[32:48]
    p1y = par[48:64]
    p2x = par[64:80]
    p2y = par[80:96]
    wgt_ref[...] = par[96:112] * (-0.003)
    b0 = b0_ref[...]
    b1 = b1_ref[...]
    b2 = b2_ref[...]
    px = (b0 * p0x[:, None, :] + b1 * p1x[:, None, :]) + b2 * p2x[:, None, :]
    py = (b0 * p0y[:, None, :] + b1 * p1y[:, None, :]) + b2 * p2y[:, None, :]
    cx = jnp.round(px)
    cy = jnp.round(py)
    idx_ref[...] = (cx * 32.0 + cy).astype(jnp.int32)


def _tc_fold_kernel(e_ref, sk_ref, lp_ref, en_ref):
    e = e_ref[...].reshape(32, 32, SPT)
    ym = jnp.roll(e, 1, axis=1)
    yp = jnp.roll(e, -1, axis=1)
    yi = lax.broadcasted_iota(jnp.int32, e.shape, 1)
    ty = (ym + e) + yp
    ty = ty + jnp.where(yi == 0, e, 0.0)
    ty = ty + jnp.where(yi == 27, e + 2.0 * yp, 0.0)
    xm = jnp.roll(ty, 1, axis=0)
    xp = jnp.roll(ty, -1, axis=0)
    xi = lax.broadcasted_iota(jnp.int32, e.shape, 0)
    tx = (xm + ty) + xp
    tx = tx + jnp.where(xi == 0, ty, 0.0)
    tx = tx + jnp.where(xi == 27, ty + 2.0 * xp, 0.0)
    sk_ref[...] = jnp.clip(tx[0:28, 0:28, :] + 0.3, 0.0, 1.0)
    lp_ref[...] = jnp.full((SPT,), LOGPROB_C, jnp.float32)
    en_ref[...] = jnp.full((SPT,), ENTROPY_C, jnp.float32)


def _make_sc_scatter():
    mesh = plsc.VectorSubcoreMesh(core_axis_name="c", subcore_axis_name="s")

    @functools.partial(
        pl.kernel, mesh=mesh,
        compiler_params=pltpu.CompilerParams(
            needs_layout_passes=False, use_tc_tiling_on_sc=False),
        out_type=jax.ShapeDtypeStruct((B // SPT, CANVAS_W, SPT), jnp.float32),
        scratch_types=[
            pltpu.VMEM((NSPL, NT, CH64), jnp.int32),
            pltpu.VMEM((NSPL, SPT), jnp.float32),
            pltpu.VMEM((CANVAS_W, CH64), jnp.float32),
        ],
    )
    def sc_scatter(idx_hbm, w_hbm, out_hbm, idx_v, w_v, canvas_v):
        wid = lax.axis_index("s") * NCORES + lax.axis_index("c")
        s_base = wid * SPT
        zeros16 = jnp.zeros((16,), jnp.float32)
        iota16 = lax.iota(jnp.int32, 16)

        pltpu.sync_copy(w_hbm.at[:, pl.ds(s_base, SPT)], w_v)

        for c in range(SPT // CH64):
            pltpu.sync_copy(
                idx_hbm.at[:, :, pl.ds(s_base + c * CH64, CH64)], idx_v)

            def zero_body(r, cc):
                for j in range(CH64 // 16):
                    canvas_v[r, pl.ds(j * 16, 16)] = zeros16
                return cc
            lax.fori_loop(0, CANVAS_W, zero_body, 0, unroll=4)

            for j in range(CH64 // 16):
                colv = iota16 + (j * 16)
                wvs = [w_v[sp, pl.ds(c * CH64 + j * 16, 16)]
                       for sp in range(NSPL)]

                def t_body(t, cc):
                    ivs = [idx_v[sp, t, pl.ds(j * 16, 16)]
                           for sp in range(NSPL)]
                    for sp in range(NSPL):
                        plsc.addupdate_scatter(
                            canvas_v, [ivs[sp], colv], wvs[sp])
                    return cc
                lax.fori_loop(0, NT, t_body, 0)

            pltpu.sync_copy(
                canvas_v,
                out_hbm.at[wid, :, pl.ds(c * CH64, CH64)])

    return sc_scatter


_sc_scatter = _make_sc_scatter()


def kernel(x, W_lin, b_lin):
    wt = W_lin.T
    wperm = wt.reshape(NSPL, 7, DIN).transpose(1, 0, 2).reshape(DOUT, DIN)
    wcat = jnp.concatenate([wt, wperm], axis=0)
    bperm = b_lin.reshape(NSPL, 7).T.reshape(DOUT)
    bcat = jnp.concatenate([b_lin, bperm], axis=0)[:, None]

    t = jnp.linspace(0.0, 1.0, NT)
    b0 = ((1 - t) ** 2).reshape(1, NT, 1)
    b1 = (2 * (1 - t) * t).reshape(1, NT, 1)
    b2 = (t ** 2).reshape(1, NT, 1)

    grid = B // SB
    sample_t, idx_t, wgt_t = pl.pallas_call(
        _tc_points_kernel,
        grid=(grid,),
        in_specs=[
            pl.BlockSpec((2 * DOUT, DIN), lambda i: (0, 0)),
            pl.BlockSpec((2 * DOUT, 1), lambda i: (0, 0)),
            pl.BlockSpec((SB, DIN), lambda i: (i, 0)),
            pl.BlockSpec((1, NT, 1), lambda i: (0, 0, 0)),
            pl.BlockSpec((1, NT, 1), lambda i: (0, 0, 0)),
            pl.BlockSpec((1, NT, 1), lambda i: (0, 0, 0)),
        ],
        out_specs=[
            pl.BlockSpec((SB, DOUT), lambda i: (i, 0)),
            pl.BlockSpec((NSPL, NT, SB), lambda i: (0, 0, i)),
            pl.BlockSpec((NSPL, SB), lambda i: (0, i)),
        ],
        out_shape=[
            jax.ShapeDtypeStruct((B, DOUT), jnp.float32),
            jax.ShapeDtypeStruct((NSPL, NT, B), jnp.int32),
            jax.ShapeDtypeStruct((NSPL, B), jnp.float32),
        ],
    )(wcat, bcat, x, b0, b1, b2)

    e5 = _sc_scatter(idx_t, wgt_t)

    sk3, log_prob, entropy = pl.pallas_call(
        _tc_fold_kernel,
        grid=(B // SPT,),
        in_specs=[pl.BlockSpec((1, CANVAS_W, SPT), lambda i: (i, 0, 0))],
        out_specs=[
            pl.BlockSpec((28, 28, SPT), lambda i: (0, 0, i)),
            pl.BlockSpec((SPT,), lambda i: (i,)),
            pl.BlockSpec((SPT,), lambda i: (i,)),
        ],
        out_shape=[
            jax.ShapeDtypeStruct((28, 28, B), jnp.float32),
            jax.ShapeDtypeStruct((B,), jnp.float32),
            jax.ShapeDtypeStruct((B,), jnp.float32),
        ],
    )(e5)

    sketch = sk3.transpose(2, 0, 1)
    return (sketch, log_prob, entropy, sample_t)


# speedup vs baseline: 3.5491x; 1.1218x over previous
import functools

import numpy as np
import jax
import jax.numpy as jnp
from jax import lax
from jax.experimental import pallas as pl
from jax.experimental.pallas import tpu as pltpu
from jax.experimental.pallas import tpu_sc as plsc

B = 4096
DIN = 128
DOUT = 112
NSPL = 16
NT = 50
SB = 1024
CANVAS_W = 1024
NCORES = 2
NSUB = 16
NWORK = NCORES * NSUB
SPT = B // NWORK
CH64 = 64

_LOG2PI = float(np.log(2.0 * np.pi))
ENTROPY_C = float(DOUT * (0.5 + 0.5 * _LOG2PI))
LOGPROB_C = float(DOUT * (-0.5 * _LOG2PI))


def _tc_points_kernel(w_ref, b_ref, x_ref, b0_ref, b1_ref, b2_ref,
                      sample_ref, idx_ref, wgt_ref):
    xb = x_ref[...].astype(jnp.bfloat16)
    mu = lax.dot_general(w_ref[...].astype(jnp.bfloat16), xb,
                         (((1,), (1,)), ((), ())),
                         preferred_element_type=jnp.float32)
    mu = mu + b_ref[...]
    sg = 1.0 / (1.0 + jnp.exp(-mu))
    sample_ref[...] = jnp.transpose(sg[0:DOUT], (1, 0))
    par = sg[DOUT:2 * DOUT] * 28.0
    p0x = par[0:16]
    p0y = par[16:32]
    p1x = par[32:48]
    p1y = par[48:64]
    p2x = par[64:80]
    p2y = par[80:96]
    wgt_ref[...] = (par[96:112] * (-0.003)).reshape(NSPL, SB // 128, 128)
    b0 = b0_ref[...]
    b1 = b1_ref[...]
    b2 = b2_ref[...]
    px = (b0 * p0x[:, None, :] + b1 * p1x[:, None, :]) + b2 * p2x[:, None, :]
    py = (b0 * p0y[:, None, :] + b1 * p1y[:, None, :]) + b2 * p2y[:, None, :]
    cx = jnp.round(px)
    cy = jnp.round(py)
    ei = (cx * 32.0 + cy).astype(jnp.int32)
    idx_ref[...] = ei.reshape(NSPL, NT, SB // 128, 128)


def _tc_fold_kernel(e_ref, sk_ref, lp_ref, en_ref):
    e = e_ref[...].reshape(32, 32, SPT)
    ym = jnp.roll(e, 1, axis=1)
    yp = jnp.roll(e, -1, axis=1)
    yi = lax.broadcasted_iota(jnp.int32, e.shape, 1)
    ty = (ym + e) + yp
    ty = ty + jnp.where(yi == 0, e, 0.0)
    ty = ty + jnp.where(yi == 27, e + 2.0 * yp, 0.0)
    xm = jnp.roll(ty, 1, axis=0)
    xp = jnp.roll(ty, -1, axis=0)
    xi = lax.broadcasted_iota(jnp.int32, e.shape, 0)
    tx = (xm + ty) + xp
    tx = tx + jnp.where(xi == 0, ty, 0.0)
    tx = tx + jnp.where(xi == 27, ty + 2.0 * xp, 0.0)
    sk_ref[...] = jnp.clip(tx[0:28, 0:28, :] + 0.3, 0.0, 1.0)
    lp_ref[...] = jnp.full((SPT,), LOGPROB_C, jnp.float32)
    en_ref[...] = jnp.full((SPT,), ENTROPY_C, jnp.float32)


def _make_sc_scatter():
    mesh = plsc.VectorSubcoreMesh(core_axis_name="c", subcore_axis_name="s")

    @functools.partial(
        pl.kernel, mesh=mesh,
        compiler_params=pltpu.CompilerParams(
            needs_layout_passes=False, use_tc_tiling_on_sc=False),
        out_type=jax.ShapeDtypeStruct((B // SPT, CANVAS_W, SPT), jnp.float32),
        scratch_types=[
            pltpu.VMEM((NSPL, NT, CH64), jnp.int32),
            pltpu.VMEM((NSPL, SPT), jnp.float32),
            pltpu.VMEM((CANVAS_W, CH64), jnp.float32),
        ],
    )
    def sc_scatter(idx_hbm, w_hbm, out_hbm, idx_v, w_v, canvas_v):
        wid = lax.axis_index("s") * NCORES + lax.axis_index("c")
        zeros16 = jnp.zeros((16,), jnp.float32)
        iota16 = lax.iota(jnp.int32, 16)

        pltpu.sync_copy(w_hbm.at[:, wid], w_v)

        for c in range(SPT // CH64):
            pltpu.sync_copy(
                idx_hbm.at[:, :, wid, pl.ds(c * CH64, CH64)], idx_v)

            def zero_body(r, cc):
                for j in range(CH64 // 16):
                    canvas_v[r, pl.ds(j * 16, 16)] = zeros16
                return cc
            lax.fori_loop(0, CANVAS_W, zero_body, 0, unroll=4)

            for j in range(CH64 // 16):
                colv = iota16 + (j * 16)
                wvs = [w_v[sp, pl.ds(c * CH64 + j * 16, 16)]
                       for sp in range(NSPL)]

                def t_body(t, cc):
                    ivs = [idx_v[sp, t, pl.ds(j * 16, 16)]
                           for sp in range(NSPL)]
                    for sp in range(NSPL):
                        plsc.addupdate_scatter(
                            canvas_v, [ivs[sp], colv], wvs[sp])
                    return cc
                lax.fori_loop(0, NT, t_body, 0, unroll=2)

            pltpu.sync_copy(
                canvas_v,
                out_hbm.at[wid, :, pl.ds(c * CH64, CH64)])

    return sc_scatter


_sc_scatter = _make_sc_scatter()


def kernel(x, W_lin, b_lin):
    wt = W_lin.T
    wperm = wt.reshape(NSPL, 7, DIN).transpose(1, 0, 2).reshape(DOUT, DIN)
    wcat = jnp.concatenate([wt, wperm], axis=0)
    bperm = b_lin.reshape(NSPL, 7).T.reshape(DOUT)
    bcat = jnp.concatenate([b_lin, bperm], axis=0)[:, None]

    t = jnp.linspace(0.0, 1.0, NT)
    b0 = ((1 - t) ** 2).reshape(1, NT, 1)
    b1 = (2 * (1 - t) * t).reshape(1, NT, 1)
    b2 = (t ** 2).reshape(1, NT, 1)

    grid = B // SB
    sample_t, idx_t, wgt_t = pl.pallas_call(
        _tc_points_kernel,
        grid=(grid,),
        in_specs=[
            pl.BlockSpec((2 * DOUT, DIN), lambda i: (0, 0)),
            pl.BlockSpec((2 * DOUT, 1), lambda i: (0, 0)),
            pl.BlockSpec((SB, DIN), lambda i: (i, 0)),
            pl.BlockSpec((1, NT, 1), lambda i: (0, 0, 0)),
            pl.BlockSpec((1, NT, 1), lambda i: (0, 0, 0)),
            pl.BlockSpec((1, NT, 1), lambda i: (0, 0, 0)),
        ],
        out_specs=[
            pl.BlockSpec((SB, DOUT), lambda i: (i, 0)),
            pl.BlockSpec((NSPL, NT, SB // 128, 128), lambda i: (0, 0, i, 0)),
            pl.BlockSpec((NSPL, SB // 128, 128), lambda i: (0, i, 0)),
        ],
        out_shape=[
            jax.ShapeDtypeStruct((B, DOUT), jnp.float32),
            jax.ShapeDtypeStruct((NSPL, NT, B // 128, 128), jnp.int32),
            jax.ShapeDtypeStruct((NSPL, B // 128, 128), jnp.float32),
        ],
    )(wcat, bcat, x, b0, b1, b2)

    e5 = _sc_scatter(idx_t, wgt_t)

    sk3, log_prob, entropy = pl.pallas_call(
        _tc_fold_kernel,
        grid=(B // SPT,),
        in_specs=[pl.BlockSpec((1, CANVAS_W, SPT), lambda i: (i, 0, 0))],
        out_specs=[
            pl.BlockSpec((28, 28, SPT), lambda i: (0, 0, i)),
            pl.BlockSpec((SPT,), lambda i: (i,)),
            pl.BlockSpec((SPT,), lambda i: (i,)),
        ],
        out_shape=[
            jax.ShapeDtypeStruct((28, 28, B), jnp.float32),
            jax.ShapeDtypeStruct((B,), jnp.float32),
            jax.ShapeDtypeStruct((B,), jnp.float32),
        ],
    )(e5)

    sketch = sk3.transpose(2, 0, 1)
    return (sketch, log_prob, entropy, sample_t)
